# baseline (speedup 1.0000x reference)
import jax
import jax.numpy as jnp
from jax.experimental import pallas as pl
from jax.experimental.pallas import tpu as pltpu


def _dialogue_gcn_body(gf_ref, spc_ref, spr_ref, wq_ref, wk_ref, v_ref,
                       wroot_ref, brg_ref, w1_ref, w2_ref, bg_ref, w8_ref,
                       out_ref,
                       gf_v, spc_v, spr_v, wq_v, wk_v, v_v,
                       wroot_v, brg_v, w1_v, w2_v, bg_v, w8_v, sems):
    L = gf_v.shape[0]
    f32 = jnp.float32

    hbm = [gf_ref, wq_ref, wk_ref, v_ref, w8_ref,
           spc_ref, spr_ref, wroot_ref, brg_ref, w1_ref, w2_ref, bg_ref]
    vmem = [gf_v, wq_v, wk_v, v_v, w8_v,
            spc_v, spr_v, wroot_v, brg_v, w1_v, w2_v, bg_v]
    cps = [pltpu.make_async_copy(h, d, sems.at[n])
           for n, (h, d) in enumerate(zip(hbm, vmem))]
    for cp in cps:
        cp.start()
    for cp in cps[:4]:
        cp.wait()

    x = gf_v[...]
    q = jnp.dot(x, wq_v[...], preferred_element_type=f32)
    k = jnp.dot(x, wk_v[...], preferred_element_type=f32)
    t3 = jnp.tanh(k[:, None, :] + q[None, :, :])
    sT = jnp.sum(t3 * v_v[...][None, :, :], axis=-1)
    m = jnp.max(sT, axis=0, keepdims=True)
    e = jnp.exp(sT - m)
    z = jnp.sum(e, axis=0, keepdims=True)
    xn = x * (1.0 / z.reshape(L, 1))

    for cp in cps[4:]:
        cp.wait()

    sp_col = spc_v[...]
    sp_row = spr_v[...]
    jj = jax.lax.broadcasted_iota(jnp.int32, (L, L), 0)
    ii = jax.lax.broadcasted_iota(jnp.int32, (L, L), 1)
    tmT = 4 * sp_row + 2 * sp_col + (ii >= jj).astype(jnp.int32)

    zero = jnp.zeros_like(e)
    acc = jnp.zeros((L, w8_v.shape[2]), dtype=f32)
    for t in range(8):
        s_t = jnp.where(tmT == t, e, zero)
        y = jnp.dot(xn, w8_v[t], preferred_element_type=f32)
        acc = acc + jnp.dot(s_t, y, preferred_element_type=f32)

    xr = acc + jnp.dot(x, wroot_v[...], preferred_element_type=f32) + brg_v[...]
    xsum = jnp.sum(xr, axis=0, keepdims=True)
    m2 = jnp.dot(xsum, w2_v[...], preferred_element_type=f32)
    out_ref[...] = (jnp.dot(xr, w1_v[...], preferred_element_type=f32)
                    + m2 + bg_v[...])


def kernel(global_features, speaker, Wq, Wk, v_att, W_rel, W_root, b_rgcn,
           W1, W2, b_gcn):
    L, D = global_features.shape
    A = Wq.shape[1]
    H = W_root.shape[1]
    G = W1.shape[1]
    f32 = jnp.float32

    sp = speaker.astype(jnp.int32)
    w8 = jax.lax.slice(
        W_rel.reshape(64, 128, D, H), (0, 0, 0, 0), (2, 4, D, H)
    ).reshape(8, D, H)

    anyspec = pl.BlockSpec(memory_space=pl.ANY)
    out = pl.pallas_call(
        _dialogue_gcn_body,
        grid=(1,),
        in_specs=[anyspec] * 12,
        out_specs=pl.BlockSpec((L, G), lambda i: (0, 0)),
        out_shape=jax.ShapeDtypeStruct((L, G), jnp.float32),
        scratch_shapes=[
            pltpu.VMEM((L, D), f32),
            pltpu.VMEM((L, 1), jnp.int32),
            pltpu.VMEM((1, L), jnp.int32),
            pltpu.VMEM((D, A), f32),
            pltpu.VMEM((D, A), f32),
            pltpu.VMEM((1, A), f32),
            pltpu.VMEM((D, H), f32),
            pltpu.VMEM((1, H), f32),
            pltpu.VMEM((H, G), f32),
            pltpu.VMEM((H, G), f32),
            pltpu.VMEM((1, G), f32),
            pltpu.VMEM((8, D, H), f32),
            pltpu.SemaphoreType.DMA((12,)),
        ],
    )(global_features, sp.reshape(L, 1), sp.reshape(1, L), Wq, Wk,
      v_att.reshape(1, A), W_root, b_rgcn.reshape(1, H), W1, W2,
      b_gcn.reshape(1, G), w8)
    return out


# speedup vs baseline: 1.0072x; 1.0072x over previous
import jax
import jax.numpy as jnp
from jax.experimental import pallas as pl
from jax.experimental.pallas import tpu as pltpu


def _dialogue_gcn_body(gf_ref, spc_ref, spr_ref, wq_ref, wk_ref, v_ref,
                       wroot_ref, brg_ref, w1_ref, w2_ref, bg_ref, w8_ref,
                       out_ref, w8v_ref, sem):
    L = gf_ref.shape[0]
    f32 = jnp.float32

    cp = pltpu.make_async_copy(w8_ref, w8v_ref, sem)
    cp.start()

    x = gf_ref[...]
    q = jnp.dot(x, wq_ref[...], preferred_element_type=f32)
    k = jnp.dot(x, wk_ref[...], preferred_element_type=f32)
    t3 = jnp.tanh(k[:, None, :] + q[None, :, :])
    sT = jnp.sum(t3 * v_ref[...][None, :, :], axis=-1)
    m = jnp.max(sT, axis=0, keepdims=True)
    e = jnp.exp(sT - m)
    z = jnp.sum(e, axis=0, keepdims=True)
    xn = x * (1.0 / z.reshape(L, 1))

    sp_col = spc_ref[...]
    sp_row = spr_ref[...]
    jj = jax.lax.broadcasted_iota(jnp.int32, (L, L), 0)
    ii = jax.lax.broadcasted_iota(jnp.int32, (L, L), 1)
    tmT = 4 * sp_row + 2 * sp_col + (ii >= jj).astype(jnp.int32)

    cp.wait()

    zero = jnp.zeros_like(e)
    acc = jnp.zeros((L, w8v_ref.shape[2]), dtype=f32)
    for t in range(8):
        s_t = jnp.where(tmT == t, e, zero)
        y = jnp.dot(xn, w8v_ref[t], preferred_element_type=f32)
        acc = acc + jnp.dot(s_t, y, preferred_element_type=f32)

    xr = acc + jnp.dot(x, wroot_ref[...], preferred_element_type=f32) + brg_ref[...]
    xsum = jnp.sum(xr, axis=0, keepdims=True)
    m2 = jnp.dot(xsum, w2_ref[...], preferred_element_type=f32)
    out_ref[...] = (jnp.dot(xr, w1_ref[...], preferred_element_type=f32)
                    + m2 + bg_ref[...])


def kernel(global_features, speaker, Wq, Wk, v_att, W_rel, W_root, b_rgcn,
           W1, W2, b_gcn):
    L, D = global_features.shape
    A = Wq.shape[1]
    H = W_root.shape[1]
    G = W1.shape[1]
    f32 = jnp.float32

    sp = speaker.astype(jnp.int32)
    sp_col = sp.reshape(L, 1)
    sp_row = sp.reshape(1, L)
    v2 = v_att.reshape(1, A)
    brg2 = b_rgcn.reshape(1, H)
    bg2 = b_gcn.reshape(1, G)
    w8 = jax.lax.slice(
        W_rel.reshape(64, 128, D, H), (0, 0, 0, 0), (2, 4, D, H)
    ).reshape(8, D, H)

    full = lambda shape: pl.BlockSpec(shape, lambda i: tuple(0 for _ in shape))
    out = pl.pallas_call(
        _dialogue_gcn_body,
        grid=(1,),
        in_specs=[
            full((L, D)),
            full((L, 1)),
            full((1, L)),
            full((D, A)),
            full((D, A)),
            full((1, A)),
            full((D, H)),
            full((1, H)),
            full((H, G)),
            full((H, G)),
            full((1, G)),
            pl.BlockSpec(memory_space=pl.ANY),
        ],
        out_specs=full((L, G)),
        out_shape=jax.ShapeDtypeStruct((L, G), jnp.float32),
        scratch_shapes=[
            pltpu.VMEM((8, D, H), f32),
            pltpu.SemaphoreType.DMA,
        ],
    )(global_features, sp_col, sp_row, Wq, Wk, v2, W_root, brg2,
      W1, W2, bg2, w8)
    return out
